# baseline (speedup 1.0000x reference)
import jax
import jax.numpy as jnp
from jax.experimental import pallas as pl
from jax.experimental.pallas import tpu as pltpu

_NEG_SLOPE = 0.01
_F = 32
_C = 2
_G = 8
_J = 16
_T = 1024


def _leaky(h):
    return jnp.maximum(h, _NEG_SLOPE * h)


def _packed_kernel(x_ref, wk_ref, w2_ref, b2_ref, w3_ref, b30_ref,
                   b31_ref, o_ref):
    bf16 = jnp.bfloat16
    xd = x_ref[...].astype(bf16)
    xk8 = jnp.concatenate([xd[:, 8 * j:8 * (j + 1)] for j in range(_J)],
                          axis=0)
    xk = jnp.concatenate([xk8, jnp.ones((_J * _T, _G), bf16)],
                         axis=1)
    h1p = jnp.dot(xk, wk_ref[...],
                  preferred_element_type=jnp.float32)
    h1 = _leaky(h1p.astype(bf16))
    h2 = jnp.dot(h1, w2_ref[...],
                 preferred_element_type=jnp.float32)
    h2 = _leaky(h2.astype(bf16) + b2_ref[...])
    h3 = jnp.dot(h2, w3_ref[...],
                 preferred_element_type=jnp.float32)
    oc0 = jnp.concatenate([h3[j * _T:(j + 1) * _T, 0:_G]
                           for j in range(_J)], axis=1) + b30_ref[...]
    oc1 = jnp.concatenate([h3[j * _T:(j + 1) * _T, _G:2 * _G]
                           for j in range(_J)], axis=1) + b31_ref[...]
    o_ref[...] = jnp.tanh(jnp.stack([oc0, oc1], axis=1).reshape(2 * _T, 128))


@jax.jit
def kernel(x, w1p, b1p, w2p, b2p, w3p, b3p):
    B = x.shape[0]
    f32 = jnp.float32
    bf16 = jnp.bfloat16
    w1 = w1p[0, :_F].astype(f32)
    b1 = b1p[0, :_F].astype(f32)
    w2 = w2p[:_F, :_F].astype(f32)
    b2 = b2p[0, :_F].astype(f32)
    w3 = w3p[:_F, :_C].astype(f32)
    b3 = b3p[0, :_C].astype(f32)

    K = _G * _F
    gc = _G * _C
    wk = jnp.zeros((2 * _G, K), f32)
    wk = wk.at[:_G, :].set(jnp.kron(jnp.eye(_G, dtype=f32), w1[None, :]))
    wk = wk.at[_G, :].set(jnp.tile(b1, _G)).astype(bf16)
    w2bd = jnp.kron(jnp.eye(_G, dtype=f32), w2).astype(bf16)
    b2t = jnp.tile(b2, _G)[None, :].astype(bf16)
    w3bd = jnp.einsum("fc,gh->gfch", w3, jnp.eye(_G, dtype=f32))
    w3bd = w3bd.reshape(K, gc)
    w3f = jnp.zeros((K, 128), f32).at[:, :gc].set(w3bd).astype(bf16)
    b30 = jnp.broadcast_to(b3[0:1], (1, 128))
    b31 = jnp.broadcast_to(b3[1:2], (1, 128))

    L = 128 * _T
    nb = pl.cdiv(B, L)
    nb += nb % 2
    Bp = nb * L
    xf = x.reshape(-1).astype(f32)
    if Bp != B:
        xf = jnp.pad(xf, (0, Bp - B))
    xd = xf.reshape(Bp // 128, 128)

    def const(shape):
        return pl.BlockSpec(shape, lambda i: (0, 0))

    cost = pl.CostEstimate(
        flops=2 * (Bp // _G) * K * (K + 128) + 2 * Bp * _F,
        transcendentals=Bp * _C,
        bytes_accessed=4 * (Bp + K * (K + 128) + Bp * _C),
    )

    out = pl.pallas_call(
        _packed_kernel,
        out_shape=jax.ShapeDtypeStruct((2 * Bp // 128, 128), f32),
        grid_spec=pltpu.PrefetchScalarGridSpec(
            num_scalar_prefetch=0,
            grid=(nb,),
            in_specs=[
                pl.BlockSpec((_T, 128), lambda i: (i, 0)),
                const((2 * _G, K)),
                const((K, K)), const((1, K)),
                const((K, 128)),
                const((1, 128)), const((1, 128)),
            ],
            out_specs=pl.BlockSpec((2 * _T, 128), lambda i: (i, 0)),
        ),
        compiler_params=pltpu.CompilerParams(
            dimension_semantics=("parallel",),
            vmem_limit_bytes=48 * 1024 * 1024,
        ),
        cost_estimate=cost,
    )(xd, wk, w2bd, b2t, w3f, b30, b31)

    out = out.reshape(Bp // 128, _C, 128).transpose(0, 2, 1).reshape(Bp, _C)
    return out[:B]


# speedup vs baseline: 1.0097x; 1.0097x over previous
import jax
import jax.numpy as jnp
from jax.experimental import pallas as pl
from jax.experimental.pallas import tpu as pltpu

_NEG_SLOPE = 0.01
_F = 32
_C = 2
_G = 8
_J = 16
_T = 2048


def _leaky(h):
    return jnp.maximum(h, _NEG_SLOPE * h)


def _packed_kernel(x_ref, wk_ref, w2_ref, b2_ref, w3_ref, b30_ref,
                   b31_ref, o_ref):
    bf16 = jnp.bfloat16
    xd = x_ref[...].astype(bf16)
    xk8 = jnp.concatenate([xd[:, 8 * j:8 * (j + 1)] for j in range(_J)],
                          axis=0)
    xk = jnp.concatenate([xk8, jnp.ones((_J * _T, _G), bf16)],
                         axis=1)
    h1p = jnp.dot(xk, wk_ref[...],
                  preferred_element_type=jnp.float32)
    h1 = _leaky(h1p.astype(bf16))
    h2 = jnp.dot(h1, w2_ref[...],
                 preferred_element_type=jnp.float32)
    h2 = _leaky(h2.astype(bf16) + b2_ref[...])
    h3 = jnp.dot(h2, w3_ref[...],
                 preferred_element_type=jnp.float32)
    oc0 = jnp.concatenate([h3[j * _T:(j + 1) * _T, 0:_G]
                           for j in range(_J)], axis=1) + b30_ref[...]
    oc1 = jnp.concatenate([h3[j * _T:(j + 1) * _T, _G:2 * _G]
                           for j in range(_J)], axis=1) + b31_ref[...]
    o_ref[...] = jnp.tanh(jnp.stack([oc0, oc1], axis=1).reshape(2 * _T, 128))


@jax.jit
def kernel(x, w1p, b1p, w2p, b2p, w3p, b3p):
    B = x.shape[0]
    f32 = jnp.float32
    bf16 = jnp.bfloat16
    w1 = w1p[0, :_F].astype(f32)
    b1 = b1p[0, :_F].astype(f32)
    w2 = w2p[:_F, :_F].astype(f32)
    b2 = b2p[0, :_F].astype(f32)
    w3 = w3p[:_F, :_C].astype(f32)
    b3 = b3p[0, :_C].astype(f32)

    K = _G * _F
    gc = _G * _C
    wk = jnp.zeros((2 * _G, K), f32)
    wk = wk.at[:_G, :].set(jnp.kron(jnp.eye(_G, dtype=f32), w1[None, :]))
    wk = wk.at[_G, :].set(jnp.tile(b1, _G)).astype(bf16)
    w2bd = jnp.kron(jnp.eye(_G, dtype=f32), w2).astype(bf16)
    b2t = jnp.tile(b2, _G)[None, :].astype(bf16)
    w3bd = jnp.einsum("fc,gh->gfch", w3, jnp.eye(_G, dtype=f32))
    w3bd = w3bd.reshape(K, gc)
    w3f = jnp.zeros((K, 128), f32).at[:, :gc].set(w3bd).astype(bf16)
    b30 = jnp.broadcast_to(b3[0:1], (1, 128))
    b31 = jnp.broadcast_to(b3[1:2], (1, 128))

    L = 128 * _T
    nb = pl.cdiv(B, L)
    nb += nb % 2
    Bp = nb * L
    xf = x.reshape(-1).astype(f32)
    if Bp != B:
        xf = jnp.pad(xf, (0, Bp - B))
    xd = xf.reshape(Bp // 128, 128)

    def const(shape):
        return pl.BlockSpec(shape, lambda i: (0, 0))

    cost = pl.CostEstimate(
        flops=2 * (Bp // _G) * K * (K + 128) + 2 * Bp * _F,
        transcendentals=Bp * _C,
        bytes_accessed=4 * (Bp + K * (K + 128) + Bp * _C),
    )

    out = pl.pallas_call(
        _packed_kernel,
        out_shape=jax.ShapeDtypeStruct((2 * Bp // 128, 128), f32),
        grid_spec=pltpu.PrefetchScalarGridSpec(
            num_scalar_prefetch=0,
            grid=(nb,),
            in_specs=[
                pl.BlockSpec((_T, 128), lambda i: (i, 0)),
                const((2 * _G, K)),
                const((K, K)), const((1, K)),
                const((K, 128)),
                const((1, 128)), const((1, 128)),
            ],
            out_specs=pl.BlockSpec((2 * _T, 128), lambda i: (i, 0)),
        ),
        compiler_params=pltpu.CompilerParams(
            dimension_semantics=("parallel",),
            vmem_limit_bytes=48 * 1024 * 1024,
        ),
        cost_estimate=cost,
    )(xd, wk, w2bd, b2t, w3f, b30, b31)

    out = out.reshape(Bp // 128, _C, 128).transpose(0, 2, 1).reshape(Bp, _C)
    return out[:B]


# speedup vs baseline: 1.0116x; 1.0019x over previous
import jax
import jax.numpy as jnp
from jax.experimental import pallas as pl
from jax.experimental.pallas import tpu as pltpu

_NEG_SLOPE = 0.01
_F = 32
_C = 2
_G = 8
_J = 16
_T = 2048


def _leaky(h):
    return jnp.maximum(h, _NEG_SLOPE * h)


def _packed_kernel(x_ref, wk_ref, w2_ref, b2_ref, w3_ref, b30_ref,
                   b31_ref, o_ref):
    bf16 = jnp.bfloat16
    xd = x_ref[...].astype(bf16)
    xk8 = jnp.concatenate([xd[:, 8 * j:8 * (j + 1)] for j in range(_J)],
                          axis=0)
    xk = jnp.concatenate([xk8, jnp.ones((_J * _T, _G), bf16)],
                         axis=1)
    h1p = jnp.dot(xk, wk_ref[...],
                  preferred_element_type=jnp.float32)
    h1 = _leaky(h1p.astype(bf16))
    h2 = jnp.dot(h1, w2_ref[...],
                 preferred_element_type=jnp.float32)
    h2 = _leaky(h2.astype(bf16) + b2_ref[...])
    h3 = jnp.dot(h2, w3_ref[...],
                 preferred_element_type=jnp.float32)
    oc0 = jnp.concatenate([h3[j * _T:(j + 1) * _T, 0:_G]
                           for j in range(_J)], axis=1) + b30_ref[...]
    oc1 = jnp.concatenate([h3[j * _T:(j + 1) * _T, _G:2 * _G]
                           for j in range(_J)], axis=1) + b31_ref[...]
    o_ref[...] = jnp.tanh(jnp.stack([oc0, oc1], axis=1).reshape(2 * _T, 128))


@jax.jit
def kernel(x, w1p, b1p, w2p, b2p, w3p, b3p):
    B = x.shape[0]
    f32 = jnp.float32
    bf16 = jnp.bfloat16
    w1 = w1p[0, :_F].astype(f32)
    b1 = b1p[0, :_F].astype(f32)
    w2 = w2p[:_F, :_F].astype(f32)
    b2 = b2p[0, :_F].astype(f32)
    w3 = w3p[:_F, :_C].astype(f32)
    b3 = b3p[0, :_C].astype(f32)

    K = _G * _F
    gc = _G * _C
    wk = jnp.zeros((2 * _G, K), f32)
    wk = wk.at[:_G, :].set(jnp.kron(jnp.eye(_G, dtype=f32), w1[None, :]))
    wk = wk.at[_G, :].set(jnp.tile(b1, _G)).astype(bf16)
    w2bd = jnp.kron(jnp.eye(_G, dtype=f32), w2).astype(bf16)
    b2t = jnp.tile(b2, _G)[None, :].astype(bf16)
    w3bd = jnp.einsum("fc,gh->gfch", w3, jnp.eye(_G, dtype=f32))
    w3bd = w3bd.reshape(K, gc)
    w3f = w3bd.astype(bf16)
    b30 = jnp.broadcast_to(b3[0:1], (1, 128))
    b31 = jnp.broadcast_to(b3[1:2], (1, 128))

    L = 128 * _T
    nb = pl.cdiv(B, L)
    nb += nb % 2
    Bp = nb * L
    xf = x.reshape(-1).astype(f32)
    if Bp != B:
        xf = jnp.pad(xf, (0, Bp - B))
    xd = xf.reshape(Bp // 128, 128)

    def const(shape):
        return pl.BlockSpec(shape, lambda i: (0, 0))

    cost = pl.CostEstimate(
        flops=2 * (Bp // _G) * K * (K + 128) + 2 * Bp * _F,
        transcendentals=Bp * _C,
        bytes_accessed=4 * (Bp + K * (K + 128) + Bp * _C),
    )

    out = pl.pallas_call(
        _packed_kernel,
        out_shape=jax.ShapeDtypeStruct((2 * Bp // 128, 128), f32),
        grid_spec=pltpu.PrefetchScalarGridSpec(
            num_scalar_prefetch=0,
            grid=(nb,),
            in_specs=[
                pl.BlockSpec((_T, 128), lambda i: (i, 0)),
                const((2 * _G, K)),
                const((K, K)), const((1, K)),
                const((K, gc)),
                const((1, 128)), const((1, 128)),
            ],
            out_specs=pl.BlockSpec((2 * _T, 128), lambda i: (i, 0)),
        ),
        compiler_params=pltpu.CompilerParams(
            dimension_semantics=("parallel",),
            vmem_limit_bytes=48 * 1024 * 1024,
        ),
        cost_estimate=cost,
    )(xd, wk, w2bd, b2t, w3f, b30, b31)

    out = out.reshape(Bp // 128, _C, 128).transpose(0, 2, 1).reshape(Bp, _C)
    return out[:B]
